# baseline (speedup 1.0000x reference)
import functools

import jax
import jax.numpy as jnp
from jax import lax
from jax.experimental import pallas as pl
from jax.experimental.pallas import tpu as pltpu
from jax.experimental.pallas import tpu_sc as plsc

N = 10000
E = 320000
D = 128
DE = 16
H = 64
T = 8
TD = 8
G = 64
OUT = 1
NSTEPS = 3
HW = 128

NC, NS = 2, 16
NW = NC * NS
N_PAD = 10240
E_PAD = 327680
NSLICE = 4
E_SL = E_PAD // NSLICE
CH = 128
RPT = N_PAD // NS

NB = 1024
NBLK = N_PAD // NB
EB = 8192

_MESH = dict(core_axis_name="c", subcore_axis_name="s", num_cores=NC,
             num_subcores=NS)


def _make_gather(e_total):
    epw = e_total // NW
    idx_rows = epw // 128
    nchunk = epw // CH

    @functools.partial(
        pl.kernel,
        out_type=jax.ShapeDtypeStruct((e_total, HW), jnp.float32),
        mesh=plsc.VectorSubcoreMesh(**_MESH),
        scratch_types=[
            pltpu.VMEM((idx_rows, 128), jnp.int32),
            pltpu.VMEM((CH, HW), jnp.float32),
            pltpu.VMEM((CH, HW), jnp.float32),
            pltpu.VMEM_SHARED((N_PAD, HW), jnp.float32),
            pltpu.SemaphoreType.DMA,
            pltpu.SemaphoreType.DMA,
            pltpu.SemaphoreType.DMA,
            pltpu.SemaphoreType.DMA,
        ],
    )
    def gather(table_hbm, idx_hbm, out_hbm, idx_v, r0, r1, tbl,
               gs0, gs1, ss0, ss1):
        cid = lax.axis_index("c")
        sid = lax.axis_index("s")
        wid = sid * NC + cid
        base = wid * epw
        pltpu.sync_copy(table_hbm.at[pl.ds(sid * RPT, RPT)],
                        tbl.at[pl.ds(sid * RPT, RPT)])
        pltpu.sync_copy(idx_hbm.at[wid], idx_v)
        plsc.subcore_barrier()
        rows = (r0, r1)
        gsem = (gs0, gs1)
        ssem = (ss0, ss1)

        def issue_gather(ci, b):
            pltpu.async_copy(tbl.at[idx_v.at[ci]], rows[b], gsem[b])

        def wait_gather(b):
            pltpu.make_async_copy(tbl.at[pl.ds(0, CH)], rows[b],
                                  gsem[b]).wait()

        def issue_store(ci, b):
            pltpu.async_copy(rows[b], out_hbm.at[pl.ds(base + ci * CH, CH)],
                             ssem[b])

        def wait_store(b):
            pltpu.make_async_copy(rows[b], out_hbm.at[pl.ds(base, CH)],
                                  ssem[b]).wait()

        issue_gather(0, 0)
        issue_gather(1, 1)
        wait_gather(0)
        issue_store(0, 0)

        def body(k, carry):
            ci = 2 * k
            wait_store(0)
            issue_gather(ci, 0)
            wait_gather(1)
            issue_store(ci - 1, 1)
            wait_store(1)
            issue_gather(ci + 1, 1)
            wait_gather(0)
            issue_store(ci, 0)
            return carry

        lax.fori_loop(1, nchunk // 2, body, 0)
        wait_store(0)
        wait_gather(1)
        issue_store(nchunk - 1, 1)
        wait_store(1)

    return gather


def _make_scatter(e_total):
    epw = e_total // NW
    idx_rows = epw // 128
    nchunks = epw // CH

    @functools.partial(
        pl.kernel,
        out_type=jax.ShapeDtypeStruct((NC, N_PAD, HW), jnp.float32),
        mesh=plsc.VectorSubcoreMesh(**_MESH),
        scratch_types=[
            pltpu.VMEM((idx_rows, 128), jnp.int32),
            pltpu.VMEM((CH, HW), jnp.float32),
            pltpu.VMEM((CH, HW), jnp.float32),
            pltpu.VMEM_SHARED((N_PAD, HW), jnp.float32),
            pltpu.SemaphoreType.DMA,
            pltpu.SemaphoreType.DMA,
            pltpu.SemaphoreType.DMA,
            pltpu.SemaphoreType.DMA,
        ],
    )
    def scatter(rows_hbm, idx_hbm, zeros_hbm, out_hbm, idx_v, r0, r1, acc,
                ls0, ls1, as0, as1):
        cid = lax.axis_index("c")
        sid = lax.axis_index("s")
        wid = sid * NC + cid
        base = wid * epw
        pltpu.sync_copy(zeros_hbm.at[pl.ds(sid * RPT, RPT)],
                        acc.at[pl.ds(sid * RPT, RPT)])
        pltpu.sync_copy(idx_hbm.at[wid], idx_v)
        plsc.subcore_barrier()
        rows = (r0, r1)
        lsem = (ls0, ls1)
        asem = (as0, as1)

        def issue_load(ci, b):
            pltpu.async_copy(rows_hbm.at[pl.ds(base + ci * CH, CH)],
                             rows[b], lsem[b])

        def wait_load(b):
            pltpu.make_async_copy(rows_hbm.at[pl.ds(base, CH)], rows[b],
                                  lsem[b]).wait()

        def issue_add(ci, b):
            pltpu.async_copy(rows[b], acc.at[idx_v.at[ci]], asem[b],
                             add=True)

        def wait_add(b):
            pltpu.make_async_copy(rows[b], acc.at[idx_v.at[0]],
                                  asem[b]).wait()

        issue_load(0, 0)
        issue_load(1, 1)
        wait_load(0)
        issue_add(0, 0)

        def chunk(k, carry):
            ci = 2 * k
            wait_add(0)
            issue_load(ci, 0)
            wait_load(1)
            issue_add(ci - 1, 1)
            wait_add(1)
            issue_load(ci + 1, 1)
            wait_load(0)
            issue_add(ci, 0)
            return carry

        lax.fori_loop(1, nchunks // 2, chunk, 0)
        wait_add(0)
        wait_load(1)
        issue_add(nchunks - 1, 1)
        wait_add(1)
        plsc.subcore_barrier()
        pltpu.sync_copy(acc.at[pl.ds(sid * RPT, RPT)],
                        out_hbm.at[cid, pl.ds(sid * RPT, RPT)])

    return scatter


_sc_gather = _make_gather(E_SL)
_sc_scatter = _make_scatter(E_SL)


def _pad_lanes(v, nrows):
    return jnp.concatenate(
        [v, jnp.zeros((nrows, HW - H), jnp.float32)], axis=1)


def _node_in_body(x_ref, w_ref, b_ref, wblk_ref, h_ref, hb_ref):
    h = jnp.maximum(
        jnp.dot(x_ref[...], w_ref[...], preferred_element_type=jnp.float32)
        + b_ref[...], 0.0)
    h_ref[...] = h
    hb_ref[...] = _pad_lanes(
        jnp.dot(h, wblk_ref[...], preferred_element_type=jnp.float32), NB)


def _edge_body(g_ref, ef_ref, we_ref, bm_ref, wmix_ref, bmix_ref, out_ref):
    pre = (g_ref[:, :H]
           + jnp.dot(ef_ref[...], we_ref[...],
                     preferred_element_type=jnp.float32)
           + bm_ref[...])
    m = jnp.maximum(pre, 0.0)
    m2 = jnp.maximum(
        jnp.dot(m, wmix_ref[...], preferred_element_type=jnp.float32)
        + bmix_ref[...], 0.0)
    out_ref[...] = _pad_lanes(m2, EB)


def _update_body(h_ref, a0_ref, a1_ref, a2_ref, a3_ref, a4_ref, a5_ref,
                 a6_ref, a7_ref, wu1_ref, wu2_ref, bu_ref, wblk_ref,
                 hout_ref, hb_ref):
    a = ((a0_ref[:, :H] + a1_ref[:, :H]) + (a2_ref[:, :H] + a3_ref[:, :H])
         + (a4_ref[:, :H] + a5_ref[:, :H])
         + (a6_ref[:, :H] + a7_ref[:, :H]))
    hn = jnp.maximum(
        jnp.dot(h_ref[...], wu1_ref[...], preferred_element_type=jnp.float32)
        + jnp.dot(a, wu2_ref[...], preferred_element_type=jnp.float32)
        + bu_ref[...], 0.0)
    rid = (pl.program_id(0) * NB
           + jax.lax.broadcasted_iota(jnp.int32, (NB, 1), 0))
    hn = jnp.where(rid < N, hn, 0.0)
    hout_ref[...] = hn
    hb_ref[...] = _pad_lanes(
        jnp.dot(hn, wblk_ref[...], preferred_element_type=jnp.float32), NB)


def _readout_body(h_ref, x_ref, b_ref, wr1h_ref, wr1x_ref, br1_ref,
                  wr2_ref, br2_ref, out_ref, acc):
    i = pl.program_id(0)

    @pl.when(i == 0)
    def _():
        acc[...] = jnp.zeros_like(acc)

    z = jnp.maximum(
        jnp.dot(h_ref[...], wr1h_ref[...], preferred_element_type=jnp.float32)
        + jnp.dot(x_ref[...], wr1x_ref[...],
                  preferred_element_type=jnp.float32)
        + br1_ref[...], 0.0)
    gids = jax.lax.broadcasted_iota(jnp.int32, (NB, G), 1)
    oh = (b_ref[...] == gids).astype(jnp.float32)
    acc[...] += lax.dot_general(oh, z, (((0,), (0,)), ((), ())),
                                preferred_element_type=jnp.float32)

    @pl.when(i == NBLK - 1)
    def _():
        out_ref[...] = (
            jnp.dot(acc[...], wr2_ref[...],
                    preferred_element_type=jnp.float32) + br2_ref[...])


def _full(shape):
    return pl.BlockSpec(shape, lambda i: tuple(0 for _ in shape))


_node_in = pl.pallas_call(
    _node_in_body,
    grid=(NBLK,),
    in_specs=[
        pl.BlockSpec((NB, D), lambda i: (i, 0)),
        _full((D, H)), _full((1, H)), _full((H, H)),
    ],
    out_specs=[pl.BlockSpec((NB, H), lambda i: (i, 0)),
               pl.BlockSpec((NB, HW), lambda i: (i, 0))],
    out_shape=[jax.ShapeDtypeStruct((N_PAD, H), jnp.float32),
               jax.ShapeDtypeStruct((N_PAD, HW), jnp.float32)],
)

def _make_edge_tc(slice_idx):
    eblk = E_SL // EB
    return pl.pallas_call(
        _edge_body,
        grid=(eblk,),
        in_specs=[
            pl.BlockSpec((EB, HW), lambda i: (i, 0)),
            pl.BlockSpec((EB, DE),
                         lambda i, s=slice_idx: (s * eblk + i, 0)),
            _full((DE, H)), _full((1, H)), _full((H, H)), _full((1, H)),
        ],
        out_specs=pl.BlockSpec((EB, HW), lambda i: (i, 0)),
        out_shape=jax.ShapeDtypeStruct((E_SL, HW), jnp.float32),
    )


_edge_tcs = [_make_edge_tc(s) for s in range(NSLICE)]

_update_tc = pl.pallas_call(
    _update_body,
    grid=(NBLK,),
    in_specs=[pl.BlockSpec((NB, H), lambda i: (i, 0))]
    + [pl.BlockSpec((NB, HW), lambda i: (i, 0))] * (2 * NSLICE)
    + [_full((H, H)), _full((H, H)), _full((1, H)), _full((H, H))],
    out_specs=[pl.BlockSpec((NB, H), lambda i: (i, 0)),
               pl.BlockSpec((NB, HW), lambda i: (i, 0))],
    out_shape=[jax.ShapeDtypeStruct((N_PAD, H), jnp.float32),
               jax.ShapeDtypeStruct((N_PAD, HW), jnp.float32)],
)

_readout_tc = pl.pallas_call(
    _readout_body,
    grid=(NBLK,),
    in_specs=[
        pl.BlockSpec((NB, H), lambda i: (i, 0)),
        pl.BlockSpec((NB, D), lambda i: (i, 0)),
        pl.BlockSpec((NB, 1), lambda i: (i, 0)),
        _full((H, H)), _full((D, H)), _full((1, H)),
        _full((H, OUT)), _full((1, OUT)),
    ],
    out_specs=_full((G, OUT)),
    out_shape=jax.ShapeDtypeStruct((G, OUT), jnp.float32),
    scratch_shapes=[pltpu.VMEM((G, H), jnp.float32)],
)


def kernel(node_features, edge_features, edge_index, batch_vector,
           W_in, b_in, W_msg, b_msg, W_mix, b_mix, W_upd, b_upd,
           W_r1, b_r1, W_r2, b_r2):
    f32 = jnp.float32
    eye = jnp.eye(T, dtype=f32)
    W_blk = (eye[:, None, :, None] * W_msg[:, :TD, None, :]).reshape(H, H)
    W_e = W_msg[:, TD:, :].transpose(1, 0, 2).reshape(DE, H)
    bm = b_msg.reshape(1, H)
    Wu1, Wu2 = W_upd[:H], W_upd[H:]
    Wr1h, Wr1x = W_r1[:H], W_r1[H:]

    x_pad = jnp.zeros((N_PAD, D), f32).at[:N].set(node_features)
    src = edge_index[0].astype(jnp.int32)
    dst = edge_index[1].astype(jnp.int32)
    src_pad = jnp.zeros((E_PAD,), jnp.int32).at[:E].set(src)
    dst_pad = jnp.full((E_PAD,), N, jnp.int32).at[:E].set(dst)
    epw = E_SL // NW
    src_sl = [src_pad[s * E_SL:(s + 1) * E_SL].reshape(NW, epw // 128, 128)
              for s in range(NSLICE)]
    dst_sl = [dst_pad[s * E_SL:(s + 1) * E_SL].reshape(NW, epw // 128, 128)
              for s in range(NSLICE)]
    batch_pad = jnp.full((N_PAD, 1), G, jnp.int32).at[:N, 0].set(
        batch_vector.astype(jnp.int32))
    zeros_nh = jnp.zeros((N_PAD, HW), f32)

    b_in2 = b_in.reshape(1, H)
    bmix = b_mix.reshape(1, H)
    bu = b_upd.reshape(1, H)
    br1 = b_r1.reshape(1, H)
    br2 = b_r2.reshape(1, OUT)

    h, hb = _node_in(x_pad, W_in, b_in2, W_blk)
    for _ in range(NSTEPS):
        aggs = []
        for s in range(NSLICE):
            g = _sc_gather(hb, src_sl[s])
            m2 = _edge_tcs[s](g, edge_features, W_e, bm, W_mix, bmix)
            aggs.append(_sc_scatter(m2, dst_sl[s], zeros_nh))
        h, hb = _update_tc(
            h, *[aggs[s][c] for s in range(NSLICE) for c in range(NC)],
            Wu1, Wu2, bu, W_blk)
    out = _readout_tc(h, x_pad, batch_pad, Wr1h, Wr1x, br1, W_r2, br2)
    return out


# speedup vs baseline: 1.1363x; 1.1363x over previous
import functools

import jax
import jax.numpy as jnp
from jax import lax
from jax.experimental import pallas as pl
from jax.experimental.pallas import tpu as pltpu
from jax.experimental.pallas import tpu_sc as plsc

N = 10000
E = 320000
D = 128
DE = 16
H = 64
T = 8
TD = 8
G = 64
OUT = 1
NSTEPS = 3
HW = 128

NC, NS = 2, 16
NW = NC * NS
N_PAD = 10240
E_PAD = 327680
NSLICE = 2
E_SL = E_PAD // NSLICE
CH = 128
RPT = N_PAD // NS

NB = 1024
NBLK = N_PAD // NB
EB = 8192

_MESH = dict(core_axis_name="c", subcore_axis_name="s", num_cores=NC,
             num_subcores=NS)


def _make_gather(e_total):
    epw = e_total // NW
    idx_rows = epw // 128
    nchunk = epw // CH

    @functools.partial(
        pl.kernel,
        out_type=jax.ShapeDtypeStruct((e_total, HW), jnp.float32),
        mesh=plsc.VectorSubcoreMesh(**_MESH),
        scratch_types=[
            pltpu.VMEM((idx_rows, 128), jnp.int32),
            pltpu.VMEM((CH, HW), jnp.float32),
            pltpu.VMEM((CH, HW), jnp.float32),
            pltpu.VMEM_SHARED((N_PAD, HW), jnp.float32),
            pltpu.SemaphoreType.DMA,
            pltpu.SemaphoreType.DMA,
            pltpu.SemaphoreType.DMA,
            pltpu.SemaphoreType.DMA,
        ],
    )
    def gather(table_hbm, idx_hbm, out_hbm, idx_v, r0, r1, tbl,
               gs0, gs1, ss0, ss1):
        cid = lax.axis_index("c")
        sid = lax.axis_index("s")
        wid = sid * NC + cid
        base = wid * epw
        pltpu.sync_copy(table_hbm.at[pl.ds(sid * RPT, RPT)],
                        tbl.at[pl.ds(sid * RPT, RPT)])
        pltpu.sync_copy(idx_hbm.at[wid], idx_v)
        plsc.subcore_barrier()
        rows = (r0, r1)
        gsem = (gs0, gs1)
        ssem = (ss0, ss1)

        def issue_gather(ci, b):
            pltpu.async_copy(tbl.at[idx_v.at[ci]], rows[b], gsem[b])

        def wait_gather(b):
            pltpu.make_async_copy(tbl.at[pl.ds(0, CH)], rows[b],
                                  gsem[b]).wait()

        def issue_store(ci, b):
            pltpu.async_copy(rows[b], out_hbm.at[pl.ds(base + ci * CH, CH)],
                             ssem[b])

        def wait_store(b):
            pltpu.make_async_copy(rows[b], out_hbm.at[pl.ds(base, CH)],
                                  ssem[b]).wait()

        issue_gather(0, 0)
        issue_gather(1, 1)
        wait_gather(0)
        issue_store(0, 0)

        def body(k, carry):
            ci = 2 * k
            wait_store(0)
            issue_gather(ci, 0)
            wait_gather(1)
            issue_store(ci - 1, 1)
            wait_store(1)
            issue_gather(ci + 1, 1)
            wait_gather(0)
            issue_store(ci, 0)
            return carry

        lax.fori_loop(1, nchunk // 2, body, 0)
        wait_store(0)
        wait_gather(1)
        issue_store(nchunk - 1, 1)
        wait_store(1)

    return gather


def _make_scatter(e_total):
    epw = e_total // NW
    idx_rows = epw // 128
    nchunks = epw // CH

    @functools.partial(
        pl.kernel,
        out_type=jax.ShapeDtypeStruct((NC, N_PAD, HW), jnp.float32),
        mesh=plsc.VectorSubcoreMesh(**_MESH),
        scratch_types=[
            pltpu.VMEM((idx_rows, 128), jnp.int32),
            pltpu.VMEM((CH, HW), jnp.float32),
            pltpu.VMEM((CH, HW), jnp.float32),
            pltpu.VMEM_SHARED((N_PAD, HW), jnp.float32),
            pltpu.SemaphoreType.DMA,
            pltpu.SemaphoreType.DMA,
            pltpu.SemaphoreType.DMA,
            pltpu.SemaphoreType.DMA,
        ],
    )
    def scatter(rows_hbm, idx_hbm, zeros_hbm, out_hbm, idx_v, r0, r1, acc,
                ls0, ls1, as0, as1):
        cid = lax.axis_index("c")
        sid = lax.axis_index("s")
        wid = sid * NC + cid
        base = wid * epw
        pltpu.sync_copy(zeros_hbm.at[pl.ds(sid * RPT, RPT)],
                        acc.at[pl.ds(sid * RPT, RPT)])
        pltpu.sync_copy(idx_hbm.at[wid], idx_v)
        plsc.subcore_barrier()
        rows = (r0, r1)
        lsem = (ls0, ls1)
        asem = (as0, as1)

        def issue_load(ci, b):
            pltpu.async_copy(rows_hbm.at[pl.ds(base + ci * CH, CH)],
                             rows[b], lsem[b])

        def wait_load(b):
            pltpu.make_async_copy(rows_hbm.at[pl.ds(base, CH)], rows[b],
                                  lsem[b]).wait()

        def issue_add(ci, b):
            pltpu.async_copy(rows[b], acc.at[idx_v.at[ci]], asem[b],
                             add=True)

        def wait_add(b):
            pltpu.make_async_copy(rows[b], acc.at[idx_v.at[0]],
                                  asem[b]).wait()

        issue_load(0, 0)
        issue_load(1, 1)
        wait_load(0)
        issue_add(0, 0)

        def chunk(k, carry):
            ci = 2 * k
            wait_add(0)
            issue_load(ci, 0)
            wait_load(1)
            issue_add(ci - 1, 1)
            wait_add(1)
            issue_load(ci + 1, 1)
            wait_load(0)
            issue_add(ci, 0)
            return carry

        lax.fori_loop(1, nchunks // 2, chunk, 0)
        wait_add(0)
        wait_load(1)
        issue_add(nchunks - 1, 1)
        wait_add(1)
        plsc.subcore_barrier()
        pltpu.sync_copy(acc.at[pl.ds(sid * RPT, RPT)],
                        out_hbm.at[cid, pl.ds(sid * RPT, RPT)])

    return scatter


_sc_gather = _make_gather(E_SL)
_sc_scatter = _make_scatter(E_SL)


def _pad_lanes(v, nrows):
    return jnp.concatenate(
        [v, jnp.zeros((nrows, HW - H), jnp.float32)], axis=1)


def _node_in_body(x_ref, w_ref, b_ref, wblk_ref, h_ref, hb_ref):
    h = jnp.maximum(
        jnp.dot(x_ref[...], w_ref[...], preferred_element_type=jnp.float32)
        + b_ref[...], 0.0)
    h_ref[...] = h
    hb_ref[...] = _pad_lanes(
        jnp.dot(h, wblk_ref[...], preferred_element_type=jnp.float32), NB)


def _edge_body(g_ref, ef_ref, we_ref, bm_ref, wmix_ref, bmix_ref, out_ref):
    pre = (g_ref[:, :H]
           + jnp.dot(ef_ref[...], we_ref[...],
                     preferred_element_type=jnp.float32)
           + bm_ref[...])
    m = jnp.maximum(pre, 0.0)
    m2 = jnp.maximum(
        jnp.dot(m, wmix_ref[...], preferred_element_type=jnp.float32)
        + bmix_ref[...], 0.0)
    out_ref[...] = _pad_lanes(m2, EB)


def _update_body(h_ref, a0_ref, a1_ref, a2_ref, a3_ref, wu1_ref, wu2_ref,
                 bu_ref, wblk_ref, hout_ref, hb_ref):
    a = (a0_ref[:, :H] + a1_ref[:, :H]) + (a2_ref[:, :H] + a3_ref[:, :H])
    hn = jnp.maximum(
        jnp.dot(h_ref[...], wu1_ref[...], preferred_element_type=jnp.float32)
        + jnp.dot(a, wu2_ref[...], preferred_element_type=jnp.float32)
        + bu_ref[...], 0.0)
    rid = (pl.program_id(0) * NB
           + jax.lax.broadcasted_iota(jnp.int32, (NB, 1), 0))
    hn = jnp.where(rid < N, hn, 0.0)
    hout_ref[...] = hn
    hb_ref[...] = _pad_lanes(
        jnp.dot(hn, wblk_ref[...], preferred_element_type=jnp.float32), NB)


def _readout_body(h_ref, x_ref, b_ref, wr1h_ref, wr1x_ref, br1_ref,
                  wr2_ref, br2_ref, out_ref, acc):
    i = pl.program_id(0)

    @pl.when(i == 0)
    def _():
        acc[...] = jnp.zeros_like(acc)

    z = jnp.maximum(
        jnp.dot(h_ref[...], wr1h_ref[...], preferred_element_type=jnp.float32)
        + jnp.dot(x_ref[...], wr1x_ref[...],
                  preferred_element_type=jnp.float32)
        + br1_ref[...], 0.0)
    gids = jax.lax.broadcasted_iota(jnp.int32, (NB, G), 1)
    oh = (b_ref[...] == gids).astype(jnp.float32)
    acc[...] += lax.dot_general(oh, z, (((0,), (0,)), ((), ())),
                                preferred_element_type=jnp.float32)

    @pl.when(i == NBLK - 1)
    def _():
        out_ref[...] = (
            jnp.dot(acc[...], wr2_ref[...],
                    preferred_element_type=jnp.float32) + br2_ref[...])


def _full(shape):
    return pl.BlockSpec(shape, lambda i: tuple(0 for _ in shape))


_node_in = pl.pallas_call(
    _node_in_body,
    grid=(NBLK,),
    in_specs=[
        pl.BlockSpec((NB, D), lambda i: (i, 0)),
        _full((D, H)), _full((1, H)), _full((H, H)),
    ],
    out_specs=[pl.BlockSpec((NB, H), lambda i: (i, 0)),
               pl.BlockSpec((NB, HW), lambda i: (i, 0))],
    out_shape=[jax.ShapeDtypeStruct((N_PAD, H), jnp.float32),
               jax.ShapeDtypeStruct((N_PAD, HW), jnp.float32)],
)

def _make_edge_tc(slice_idx):
    eblk = E_SL // EB
    return pl.pallas_call(
        _edge_body,
        grid=(eblk,),
        in_specs=[
            pl.BlockSpec((EB, HW), lambda i: (i, 0)),
            pl.BlockSpec((EB, DE),
                         lambda i, s=slice_idx: (s * eblk + i, 0)),
            _full((DE, H)), _full((1, H)), _full((H, H)), _full((1, H)),
        ],
        out_specs=pl.BlockSpec((EB, HW), lambda i: (i, 0)),
        out_shape=jax.ShapeDtypeStruct((E_SL, HW), jnp.float32),
    )


_edge_tcs = [_make_edge_tc(s) for s in range(NSLICE)]

_update_tc = pl.pallas_call(
    _update_body,
    grid=(NBLK,),
    in_specs=[pl.BlockSpec((NB, H), lambda i: (i, 0))]
    + [pl.BlockSpec((NB, HW), lambda i: (i, 0))] * (2 * NSLICE)
    + [_full((H, H)), _full((H, H)), _full((1, H)), _full((H, H))],
    out_specs=[pl.BlockSpec((NB, H), lambda i: (i, 0)),
               pl.BlockSpec((NB, HW), lambda i: (i, 0))],
    out_shape=[jax.ShapeDtypeStruct((N_PAD, H), jnp.float32),
               jax.ShapeDtypeStruct((N_PAD, HW), jnp.float32)],
)

_readout_tc = pl.pallas_call(
    _readout_body,
    grid=(NBLK,),
    in_specs=[
        pl.BlockSpec((NB, H), lambda i: (i, 0)),
        pl.BlockSpec((NB, D), lambda i: (i, 0)),
        pl.BlockSpec((NB, 1), lambda i: (i, 0)),
        _full((H, H)), _full((D, H)), _full((1, H)),
        _full((H, OUT)), _full((1, OUT)),
    ],
    out_specs=_full((G, OUT)),
    out_shape=jax.ShapeDtypeStruct((G, OUT), jnp.float32),
    scratch_shapes=[pltpu.VMEM((G, H), jnp.float32)],
)


def kernel(node_features, edge_features, edge_index, batch_vector,
           W_in, b_in, W_msg, b_msg, W_mix, b_mix, W_upd, b_upd,
           W_r1, b_r1, W_r2, b_r2):
    f32 = jnp.float32
    eye = jnp.eye(T, dtype=f32)
    W_blk = (eye[:, None, :, None] * W_msg[:, :TD, None, :]).reshape(H, H)
    W_e = W_msg[:, TD:, :].transpose(1, 0, 2).reshape(DE, H)
    bm = b_msg.reshape(1, H)
    Wu1, Wu2 = W_upd[:H], W_upd[H:]
    Wr1h, Wr1x = W_r1[:H], W_r1[H:]

    x_pad = jnp.zeros((N_PAD, D), f32).at[:N].set(node_features)
    src = edge_index[0].astype(jnp.int32)
    dst = edge_index[1].astype(jnp.int32)
    src_pad = jnp.zeros((E_PAD,), jnp.int32).at[:E].set(src)
    dst_pad = jnp.full((E_PAD,), N, jnp.int32).at[:E].set(dst)
    epw = E_SL // NW
    src_sl = [src_pad[s * E_SL:(s + 1) * E_SL].reshape(NW, epw // 128, 128)
              for s in range(NSLICE)]
    dst_sl = [dst_pad[s * E_SL:(s + 1) * E_SL].reshape(NW, epw // 128, 128)
              for s in range(NSLICE)]
    batch_pad = jnp.full((N_PAD, 1), G, jnp.int32).at[:N, 0].set(
        batch_vector.astype(jnp.int32))
    zeros_nh = jnp.zeros((N_PAD, HW), f32)

    b_in2 = b_in.reshape(1, H)
    bmix = b_mix.reshape(1, H)
    bu = b_upd.reshape(1, H)
    br1 = b_r1.reshape(1, H)
    br2 = b_r2.reshape(1, OUT)

    h, hb = _node_in(x_pad, W_in, b_in2, W_blk)
    for _ in range(NSTEPS):
        aggs = []
        for s in range(NSLICE):
            g = _sc_gather(hb, src_sl[s])
            m2 = _edge_tcs[s](g, edge_features, W_e, bm, W_mix, bmix)
            aggs.append(_sc_scatter(m2, dst_sl[s], zeros_nh))
        h, hb = _update_tc(
            h, *[aggs[s][c] for s in range(NSLICE) for c in range(NC)],
            Wu1, Wu2, bu, W_blk)
    out = _readout_tc(h, x_pad, batch_pad, Wr1h, Wr1x, br1, W_r2, br2)
    return out


# speedup vs baseline: 1.1392x; 1.0025x over previous
import functools

import jax
import jax.numpy as jnp
from jax import lax
from jax.experimental import pallas as pl
from jax.experimental.pallas import tpu as pltpu
from jax.experimental.pallas import tpu_sc as plsc

N = 10000
E = 320000
D = 128
DE = 16
H = 64
T = 8
TD = 8
G = 64
OUT = 1
NSTEPS = 3
HW = 128

NC, NS = 2, 16
NW = NC * NS
N_PAD = 10240
E_PAD = 327680
NSLICE = 2
E_SL = E_PAD // NSLICE
CH = 128
RPT = N_PAD // NS

NB = 1024
NBLK = N_PAD // NB
EB = 16384

_MESH = dict(core_axis_name="c", subcore_axis_name="s", num_cores=NC,
             num_subcores=NS)


def _make_gather(e_total):
    epw = e_total // NW
    idx_rows = epw // 128
    nchunk = epw // CH

    @functools.partial(
        pl.kernel,
        out_type=jax.ShapeDtypeStruct((e_total, HW), jnp.float32),
        mesh=plsc.VectorSubcoreMesh(**_MESH),
        scratch_types=[
            pltpu.VMEM((idx_rows, 128), jnp.int32),
            pltpu.VMEM((CH, HW), jnp.float32),
            pltpu.VMEM((CH, HW), jnp.float32),
            pltpu.VMEM_SHARED((N_PAD, HW), jnp.float32),
            pltpu.SemaphoreType.DMA,
            pltpu.SemaphoreType.DMA,
            pltpu.SemaphoreType.DMA,
            pltpu.SemaphoreType.DMA,
        ],
    )
    def gather(table_hbm, idx_hbm, out_hbm, idx_v, r0, r1, tbl,
               gs0, gs1, ss0, ss1):
        cid = lax.axis_index("c")
        sid = lax.axis_index("s")
        wid = sid * NC + cid
        base = wid * epw
        pltpu.sync_copy(table_hbm.at[pl.ds(sid * RPT, RPT)],
                        tbl.at[pl.ds(sid * RPT, RPT)])
        pltpu.sync_copy(idx_hbm.at[wid], idx_v)
        plsc.subcore_barrier()
        rows = (r0, r1)
        gsem = (gs0, gs1)
        ssem = (ss0, ss1)

        def issue_gather(ci, b):
            pltpu.async_copy(tbl.at[idx_v.at[ci]], rows[b], gsem[b])

        def wait_gather(b):
            pltpu.make_async_copy(tbl.at[pl.ds(0, CH)], rows[b],
                                  gsem[b]).wait()

        def issue_store(ci, b):
            pltpu.async_copy(rows[b], out_hbm.at[pl.ds(base + ci * CH, CH)],
                             ssem[b])

        def wait_store(b):
            pltpu.make_async_copy(rows[b], out_hbm.at[pl.ds(base, CH)],
                                  ssem[b]).wait()

        issue_gather(0, 0)
        issue_gather(1, 1)
        wait_gather(0)
        issue_store(0, 0)

        def body(k, carry):
            ci = 2 * k
            wait_store(0)
            issue_gather(ci, 0)
            wait_gather(1)
            issue_store(ci - 1, 1)
            wait_store(1)
            issue_gather(ci + 1, 1)
            wait_gather(0)
            issue_store(ci, 0)
            return carry

        lax.fori_loop(1, nchunk // 2, body, 0)
        wait_store(0)
        wait_gather(1)
        issue_store(nchunk - 1, 1)
        wait_store(1)

    return gather


def _make_scatter(e_total):
    epw = e_total // NW
    idx_rows = epw // 128
    nchunks = epw // CH

    @functools.partial(
        pl.kernel,
        out_type=jax.ShapeDtypeStruct((NC, N_PAD, HW), jnp.float32),
        mesh=plsc.VectorSubcoreMesh(**_MESH),
        scratch_types=[
            pltpu.VMEM((idx_rows, 128), jnp.int32),
            pltpu.VMEM((CH, HW), jnp.float32),
            pltpu.VMEM((CH, HW), jnp.float32),
            pltpu.VMEM_SHARED((N_PAD, HW), jnp.float32),
            pltpu.SemaphoreType.DMA,
            pltpu.SemaphoreType.DMA,
            pltpu.SemaphoreType.DMA,
            pltpu.SemaphoreType.DMA,
        ],
    )
    def scatter(rows_hbm, idx_hbm, zeros_hbm, out_hbm, idx_v, r0, r1, acc,
                ls0, ls1, as0, as1):
        cid = lax.axis_index("c")
        sid = lax.axis_index("s")
        wid = sid * NC + cid
        base = wid * epw
        pltpu.sync_copy(zeros_hbm.at[pl.ds(sid * RPT, RPT)],
                        acc.at[pl.ds(sid * RPT, RPT)])
        pltpu.sync_copy(idx_hbm.at[wid], idx_v)
        plsc.subcore_barrier()
        rows = (r0, r1)
        lsem = (ls0, ls1)
        asem = (as0, as1)

        def issue_load(ci, b):
            pltpu.async_copy(rows_hbm.at[pl.ds(base + ci * CH, CH)],
                             rows[b], lsem[b])

        def wait_load(b):
            pltpu.make_async_copy(rows_hbm.at[pl.ds(base, CH)], rows[b],
                                  lsem[b]).wait()

        def issue_add(ci, b):
            pltpu.async_copy(rows[b], acc.at[idx_v.at[ci]], asem[b],
                             add=True)

        def wait_add(b):
            pltpu.make_async_copy(rows[b], acc.at[idx_v.at[0]],
                                  asem[b]).wait()

        issue_load(0, 0)
        issue_load(1, 1)
        wait_load(0)
        issue_add(0, 0)

        def chunk(k, carry):
            ci = 2 * k
            wait_add(0)
            issue_load(ci, 0)
            wait_load(1)
            issue_add(ci - 1, 1)
            wait_add(1)
            issue_load(ci + 1, 1)
            wait_load(0)
            issue_add(ci, 0)
            return carry

        lax.fori_loop(1, nchunks // 2, chunk, 0)
        wait_add(0)
        wait_load(1)
        issue_add(nchunks - 1, 1)
        wait_add(1)
        plsc.subcore_barrier()
        pltpu.sync_copy(acc.at[pl.ds(sid * RPT, RPT)],
                        out_hbm.at[cid, pl.ds(sid * RPT, RPT)])

    return scatter


_sc_gather = _make_gather(E_SL)
_sc_scatter = _make_scatter(E_SL)


def _pad_lanes(v, nrows):
    return jnp.concatenate(
        [v, jnp.zeros((nrows, HW - H), jnp.float32)], axis=1)


def _node_in_body(x_ref, w_ref, b_ref, wblk_ref, h_ref, hb_ref):
    h = jnp.maximum(
        jnp.dot(x_ref[...], w_ref[...], preferred_element_type=jnp.float32)
        + b_ref[...], 0.0)
    h_ref[...] = h
    hb_ref[...] = _pad_lanes(
        jnp.dot(h, wblk_ref[...], preferred_element_type=jnp.float32), NB)


def _edge_body(g_ref, ef_ref, we_ref, bm_ref, wmix_ref, bmix_ref, out_ref):
    pre = (g_ref[:, :H]
           + jnp.dot(ef_ref[...], we_ref[...],
                     preferred_element_type=jnp.float32)
           + bm_ref[...])
    m = jnp.maximum(pre, 0.0)
    m2 = jnp.maximum(
        jnp.dot(m, wmix_ref[...], preferred_element_type=jnp.float32)
        + bmix_ref[...], 0.0)
    out_ref[...] = _pad_lanes(m2, EB)


def _update_body(h_ref, a0_ref, a1_ref, a2_ref, a3_ref, wu1_ref, wu2_ref,
                 bu_ref, wblk_ref, hout_ref, hb_ref):
    a = (a0_ref[:, :H] + a1_ref[:, :H]) + (a2_ref[:, :H] + a3_ref[:, :H])
    hn = jnp.maximum(
        jnp.dot(h_ref[...], wu1_ref[...], preferred_element_type=jnp.float32)
        + jnp.dot(a, wu2_ref[...], preferred_element_type=jnp.float32)
        + bu_ref[...], 0.0)
    rid = (pl.program_id(0) * NB
           + jax.lax.broadcasted_iota(jnp.int32, (NB, 1), 0))
    hn = jnp.where(rid < N, hn, 0.0)
    hout_ref[...] = hn
    hb_ref[...] = _pad_lanes(
        jnp.dot(hn, wblk_ref[...], preferred_element_type=jnp.float32), NB)


def _readout_body(h_ref, x_ref, b_ref, wr1h_ref, wr1x_ref, br1_ref,
                  wr2_ref, br2_ref, out_ref, acc):
    i = pl.program_id(0)

    @pl.when(i == 0)
    def _():
        acc[...] = jnp.zeros_like(acc)

    z = jnp.maximum(
        jnp.dot(h_ref[...], wr1h_ref[...], preferred_element_type=jnp.float32)
        + jnp.dot(x_ref[...], wr1x_ref[...],
                  preferred_element_type=jnp.float32)
        + br1_ref[...], 0.0)
    gids = jax.lax.broadcasted_iota(jnp.int32, (NB, G), 1)
    oh = (b_ref[...] == gids).astype(jnp.float32)
    acc[...] += lax.dot_general(oh, z, (((0,), (0,)), ((), ())),
                                preferred_element_type=jnp.float32)

    @pl.when(i == NBLK - 1)
    def _():
        out_ref[...] = (
            jnp.dot(acc[...], wr2_ref[...],
                    preferred_element_type=jnp.float32) + br2_ref[...])


def _full(shape):
    return pl.BlockSpec(shape, lambda i: tuple(0 for _ in shape))


_node_in = pl.pallas_call(
    _node_in_body,
    grid=(NBLK,),
    in_specs=[
        pl.BlockSpec((NB, D), lambda i: (i, 0)),
        _full((D, H)), _full((1, H)), _full((H, H)),
    ],
    out_specs=[pl.BlockSpec((NB, H), lambda i: (i, 0)),
               pl.BlockSpec((NB, HW), lambda i: (i, 0))],
    out_shape=[jax.ShapeDtypeStruct((N_PAD, H), jnp.float32),
               jax.ShapeDtypeStruct((N_PAD, HW), jnp.float32)],
)

def _make_edge_tc(slice_idx):
    eblk = E_SL // EB
    return pl.pallas_call(
        _edge_body,
        grid=(eblk,),
        in_specs=[
            pl.BlockSpec((EB, HW), lambda i: (i, 0)),
            pl.BlockSpec((EB, DE),
                         lambda i, s=slice_idx: (s * eblk + i, 0)),
            _full((DE, H)), _full((1, H)), _full((H, H)), _full((1, H)),
        ],
        out_specs=pl.BlockSpec((EB, HW), lambda i: (i, 0)),
        out_shape=jax.ShapeDtypeStruct((E_SL, HW), jnp.float32),
    )


_edge_tcs = [_make_edge_tc(s) for s in range(NSLICE)]

_update_tc = pl.pallas_call(
    _update_body,
    grid=(NBLK,),
    in_specs=[pl.BlockSpec((NB, H), lambda i: (i, 0))]
    + [pl.BlockSpec((NB, HW), lambda i: (i, 0))] * (2 * NSLICE)
    + [_full((H, H)), _full((H, H)), _full((1, H)), _full((H, H))],
    out_specs=[pl.BlockSpec((NB, H), lambda i: (i, 0)),
               pl.BlockSpec((NB, HW), lambda i: (i, 0))],
    out_shape=[jax.ShapeDtypeStruct((N_PAD, H), jnp.float32),
               jax.ShapeDtypeStruct((N_PAD, HW), jnp.float32)],
)

_readout_tc = pl.pallas_call(
    _readout_body,
    grid=(NBLK,),
    in_specs=[
        pl.BlockSpec((NB, H), lambda i: (i, 0)),
        pl.BlockSpec((NB, D), lambda i: (i, 0)),
        pl.BlockSpec((NB, 1), lambda i: (i, 0)),
        _full((H, H)), _full((D, H)), _full((1, H)),
        _full((H, OUT)), _full((1, OUT)),
    ],
    out_specs=_full((G, OUT)),
    out_shape=jax.ShapeDtypeStruct((G, OUT), jnp.float32),
    scratch_shapes=[pltpu.VMEM((G, H), jnp.float32)],
)


def kernel(node_features, edge_features, edge_index, batch_vector,
           W_in, b_in, W_msg, b_msg, W_mix, b_mix, W_upd, b_upd,
           W_r1, b_r1, W_r2, b_r2):
    f32 = jnp.float32
    eye = jnp.eye(T, dtype=f32)
    W_blk = (eye[:, None, :, None] * W_msg[:, :TD, None, :]).reshape(H, H)
    W_e = W_msg[:, TD:, :].transpose(1, 0, 2).reshape(DE, H)
    bm = b_msg.reshape(1, H)
    Wu1, Wu2 = W_upd[:H], W_upd[H:]
    Wr1h, Wr1x = W_r1[:H], W_r1[H:]

    x_pad = jnp.zeros((N_PAD, D), f32).at[:N].set(node_features)
    src = edge_index[0].astype(jnp.int32)
    dst = edge_index[1].astype(jnp.int32)
    src_pad = jnp.zeros((E_PAD,), jnp.int32).at[:E].set(src)
    dst_pad = jnp.full((E_PAD,), N, jnp.int32).at[:E].set(dst)
    epw = E_SL // NW
    src_sl = [src_pad[s * E_SL:(s + 1) * E_SL].reshape(NW, epw // 128, 128)
              for s in range(NSLICE)]
    dst_sl = [dst_pad[s * E_SL:(s + 1) * E_SL].reshape(NW, epw // 128, 128)
              for s in range(NSLICE)]
    batch_pad = jnp.full((N_PAD, 1), G, jnp.int32).at[:N, 0].set(
        batch_vector.astype(jnp.int32))
    zeros_nh = jnp.zeros((N_PAD, HW), f32)

    b_in2 = b_in.reshape(1, H)
    bmix = b_mix.reshape(1, H)
    bu = b_upd.reshape(1, H)
    br1 = b_r1.reshape(1, H)
    br2 = b_r2.reshape(1, OUT)

    h, hb = _node_in(x_pad, W_in, b_in2, W_blk)
    for _ in range(NSTEPS):
        aggs = []
        for s in range(NSLICE):
            g = _sc_gather(hb, src_sl[s])
            m2 = _edge_tcs[s](g, edge_features, W_e, bm, W_mix, bmix)
            aggs.append(_sc_scatter(m2, dst_sl[s], zeros_nh))
        h, hb = _update_tc(
            h, *[aggs[s][c] for s in range(NSLICE) for c in range(NC)],
            Wu1, Wu2, bu, W_blk)
    out = _readout_tc(h, x_pad, batch_pad, Wr1h, Wr1x, br1, W_r2, br2)
    return out
